# Initial kernel scaffold; baseline (speedup 1.0000x reference)
import jax
import jax.numpy as jnp
from jax.experimental import pallas as pl


def kernel(x, edge_index, W1_l, b1, W1_r, W2_l, b2, W2_r):
    raise NotImplementedError("write your pallas kernel here")


# speedup vs baseline: 2.9672x; 2.9672x over previous
import functools

import jax
import jax.numpy as jnp
from jax import lax
from jax.experimental import pallas as pl
from jax.experimental.pallas import tpu as pltpu
from jax.experimental.pallas import tpu_sc as plsc

N = 10000
D = 256
H = 128
E = 160000
NTILES = 16
CHUNK = 128
NCH = 80
EPT = NCH * CHUNK
EPAD = NTILES * EPT
NPAD = 10240
ZROWS = NPAD // NTILES
OROWS = 1000
TN = 400


def _dot_t(a, b):
    return lax.dot_general(a, b, (((1,), (1,)), ((), ())),
                           preferred_element_type=jnp.float32)


def _tc_pre_body(x_ref, wl_ref, wr_ref, b_ref, y_ref, r_ref):
    xb = x_ref[...]
    y = _dot_t(xb, wl_ref[...])
    y_ref[0] = y[:, :H]
    y_ref[1] = y[:, H:]
    r_ref[...] = _dot_t(xb, wr_ref[...]) + b_ref[...]


def _tc_pre(f, wl, wr, b):
    return pl.pallas_call(
        _tc_pre_body,
        grid=(N // TN,),
        in_specs=[
            pl.BlockSpec((TN, D), lambda i: (i, 0)),
            pl.BlockSpec((D, D), lambda i: (0, 0)),
            pl.BlockSpec((D, D), lambda i: (0, 0)),
            pl.BlockSpec((1, D), lambda i: (0, 0)),
        ],
        out_specs=[
            pl.BlockSpec((2, TN, H), lambda i: (0, i, 0)),
            pl.BlockSpec((TN, D), lambda i: (i, 0)),
        ],
        out_shape=[
            jax.ShapeDtypeStruct((2, N, H), jnp.float32),
            jax.ShapeDtypeStruct((N, D), jnp.float32),
        ],
    )(f, wl, wr, b.reshape(1, D))


def _tc_mid_body(agg_ref, cnt_ref, r_ref, wl_ref, wr_ref, b_ref,
                 y_ref, r2_ref):
    recip = 1.0 / jnp.maximum(cnt_ref[...], 1.0)
    hb = jnp.concatenate([agg_ref[0], agg_ref[1]], axis=1) * recip + r_ref[...]
    y = _dot_t(hb, wl_ref[...])
    y_ref[0] = y[:, :H]
    y_ref[1] = y[:, H:]
    r2_ref[...] = _dot_t(hb, wr_ref[...]) + b_ref[...]


def _tc_mid(agg, cnt, r, wl, wr, b):
    return pl.pallas_call(
        _tc_mid_body,
        grid=(N // TN,),
        in_specs=[
            pl.BlockSpec((2, TN, H), lambda i: (0, i, 0)),
            pl.BlockSpec((TN, 1), lambda i: (i, 0)),
            pl.BlockSpec((TN, D), lambda i: (i, 0)),
            pl.BlockSpec((D, D), lambda i: (0, 0)),
            pl.BlockSpec((D, D), lambda i: (0, 0)),
            pl.BlockSpec((1, D), lambda i: (0, 0)),
        ],
        out_specs=[
            pl.BlockSpec((2, TN, H), lambda i: (0, i, 0)),
            pl.BlockSpec((TN, D), lambda i: (i, 0)),
        ],
        out_shape=[
            jax.ShapeDtypeStruct((2, N, H), jnp.float32),
            jax.ShapeDtypeStruct((N, D), jnp.float32),
        ],
    )(agg, cnt, r, wl, wr, b.reshape(1, D))


def _tc_post_body(agg_ref, cnt_ref, r_ref, o_ref):
    recip = 1.0 / jnp.maximum(cnt_ref[...], 1.0)
    o_ref[...] = (jnp.concatenate([agg_ref[0], agg_ref[1]], axis=1) * recip
                  + r_ref[...])


def _tc_post(agg, cnt, r):
    return pl.pallas_call(
        _tc_post_body,
        grid=(N // TN,),
        in_specs=[
            pl.BlockSpec((2, TN, H), lambda i: (0, i, 0)),
            pl.BlockSpec((TN, 1), lambda i: (i, 0)),
            pl.BlockSpec((TN, D), lambda i: (i, 0)),
        ],
        out_specs=pl.BlockSpec((TN, D), lambda i: (i, 0)),
        out_shape=jax.ShapeDtypeStruct((N, D), jnp.float32),
    )(agg, cnt, r)


def _make_sc_agg(with_cnt: bool):
    mesh = plsc.VectorSubcoreMesh(core_axis_name="c", subcore_axis_name="s")
    out_type = [jax.ShapeDtypeStruct((2, N, H), jnp.float32)]
    if with_cnt:
        out_type.append(jax.ShapeDtypeStruct((N,), jnp.float32))

    def body(y_hbm, srcs_hbm, dsts_hbm, agg_hbm, *rest):
        if with_cnt:
            cnt_hbm = rest[0]
            src_v, dst_v, rows_v, ones_v, cnt_v, acc, cnt_acc, gsem = rest[1:]
        else:
            src_v, dst_v, rows_v, ones_v, cnt_v, acc, cnt_acc, gsem = rest

        c = lax.axis_index("c")
        s = lax.axis_index("s")

        zv = jnp.zeros((16,), jnp.float32)

        def zbody(i, _):
            for k in range(H // 16):
                rows_v[0, i, pl.ds(k * 16, 16)] = zv
            return 0

        lax.fori_loop(0, CHUNK, zbody, 0)
        for t in range(ZROWS // CHUNK):
            pltpu.sync_copy(rows_v.at[0],
                            acc.at[pl.ds(s * ZROWS + t * CHUNK, CHUNK)])
        if with_cnt:
            for k in range(1024 // 16):
                cnt_v[pl.ds(k * 16, 16)] = zv

            @pl.when(c == 0)
            def _():
                pltpu.sync_copy(cnt_v.at[pl.ds(0, NPAD // NTILES)],
                                cnt_acc.at[pl.ds(s * (NPAD // NTILES),
                                                 NPAD // NTILES)])

        pltpu.sync_copy(srcs_hbm.at[s], src_v)
        pltpu.sync_copy(dsts_hbm.at[s], dst_v)

        off = c * N

        def off_body(i, _):
            sl = pl.ds(i * 16, 16)
            src_v[sl] = src_v[sl] + off
            return 0

        lax.fori_loop(0, EPT // 16, off_body, 0)

        if with_cnt:
            for k in range(CHUNK // 16):
                ones_v[pl.ds(k * 16, 16)] = jnp.full((16,), 1.0, jnp.float32)

        plsc.subcore_barrier()

        def chunk_body(j, _):
            idx = src_v.at[pl.ds(j * CHUNK, CHUNK)]
            pltpu.async_copy(y_hbm.at[idx], rows_v.at[0], gsem).wait()
            pltpu.sync_copy(rows_v.at[0], acc.at[dst_v.at[j]], add=True)
            return 0

        lax.fori_loop(0, NCH, chunk_body, 0)

        if with_cnt:
            @pl.when(c == 0)
            def _():
                def cnt_body(j, _):
                    pltpu.sync_copy(ones_v, cnt_acc.at[dst_v.at[j]], add=True)
                    return 0
                lax.fori_loop(0, NCH, cnt_body, 0)

        plsc.subcore_barrier()

        @pl.when(s < 10)
        def _():
            pltpu.sync_copy(acc.at[pl.ds(s * OROWS, OROWS)],
                            agg_hbm.at[c, pl.ds(s * OROWS, OROWS)])
        if with_cnt:
            @pl.when(jnp.logical_and(c == 0, s < 10))
            def _():
                pltpu.sync_copy(cnt_acc.at[pl.ds(s * 1000, 1000)],
                                cnt_v.at[pl.ds(0, 1000)])
                pltpu.sync_copy(cnt_v.at[pl.ds(0, 1000)],
                                cnt_hbm.at[pl.ds(s * 1000, 1000)])

    return pl.kernel(
        body,
        out_type=out_type,
        mesh=mesh,
        scratch_types=[
            pltpu.VMEM((EPT,), jnp.int32),
            pltpu.VMEM((NCH, CHUNK), jnp.int32),
            pltpu.VMEM((1, CHUNK, H), jnp.float32),
            pltpu.VMEM((CHUNK,), jnp.float32),
            pltpu.VMEM((1024,), jnp.float32),
            pltpu.VMEM_SHARED((NPAD, H), jnp.float32),
            pltpu.VMEM_SHARED((NPAD,), jnp.float32),
            pltpu.SemaphoreType.DMA,
        ],
    )


_sc_agg_cnt = _make_sc_agg(True)
_sc_agg = _make_sc_agg(False)


def kernel(x, edge_index, W1_l, b1, W1_r, W2_l, b2, W2_r):
    src = edge_index[0].astype(jnp.int32)
    dst = edge_index[1].astype(jnp.int32)
    pad = EPAD - E
    src_p = jnp.concatenate([src, jnp.zeros((pad,), jnp.int32)])
    dst_p = jnp.concatenate([dst, jnp.full((pad,), N, jnp.int32)])
    srcs = src_p.reshape(NTILES, EPT)
    dsts = dst_p.reshape(NTILES, NCH, CHUNK)

    y1, r1 = _tc_pre(x, W1_l, W1_r, b1)
    agg1, cnt = _sc_agg_cnt(y1.reshape(2 * N, H), srcs, dsts)
    cnt2 = cnt.reshape(N, 1)
    y2, r2 = _tc_mid(agg1, cnt2, r1, W2_l, W2_r, b2)
    (agg2,) = _sc_agg(y2.reshape(2 * N, H), srcs, dsts)
    return _tc_post(agg2, cnt2, r2)


# speedup vs baseline: 3.9256x; 1.3230x over previous
import functools

import jax
import jax.numpy as jnp
from jax import lax
from jax.experimental import pallas as pl
from jax.experimental.pallas import tpu as pltpu
from jax.experimental.pallas import tpu_sc as plsc

N = 10000
D = 256
H = 128
E = 160000
NTILES = 16
CHUNK = 64
NCH = 158
EPT = NCH * CHUNK
EPAD = NTILES * EPT
NPAD = 10240
ZROWS = NPAD // NTILES
OROWS = 1000
TN = 400


def _dot_t(a, b):
    return lax.dot_general(a, b, (((1,), (1,)), ((), ())),
                           preferred_element_type=jnp.float32)


def _tc_pre_body(x_ref, wl_ref, wr_ref, b_ref, y_ref, r_ref):
    xb = x_ref[...]
    y = _dot_t(xb, wl_ref[...])
    y_ref[0] = y[:, :H]
    y_ref[1] = y[:, H:]
    r_ref[...] = _dot_t(xb, wr_ref[...]) + b_ref[...]


def _tc_pre(f, wl, wr, b):
    return pl.pallas_call(
        _tc_pre_body,
        grid=(N // TN,),
        in_specs=[
            pl.BlockSpec((TN, D), lambda i: (i, 0)),
            pl.BlockSpec((D, D), lambda i: (0, 0)),
            pl.BlockSpec((D, D), lambda i: (0, 0)),
            pl.BlockSpec((1, D), lambda i: (0, 0)),
        ],
        out_specs=[
            pl.BlockSpec((2, TN, H), lambda i: (0, i, 0)),
            pl.BlockSpec((TN, D), lambda i: (i, 0)),
        ],
        out_shape=[
            jax.ShapeDtypeStruct((2, N, H), jnp.float32),
            jax.ShapeDtypeStruct((N, D), jnp.float32),
        ],
    )(f, wl, wr, b.reshape(1, D))


def _tc_mid_body(agg_ref, cnt_ref, r_ref, wl_ref, wr_ref, b_ref,
                 y_ref, r2_ref):
    recip = 1.0 / jnp.maximum(cnt_ref[...], 1.0)
    hb = jnp.concatenate([agg_ref[0], agg_ref[1]], axis=1) * recip + r_ref[...]
    y = _dot_t(hb, wl_ref[...])
    y_ref[0] = y[:, :H]
    y_ref[1] = y[:, H:]
    r2_ref[...] = _dot_t(hb, wr_ref[...]) + b_ref[...]


def _tc_mid(agg, cnt, r, wl, wr, b):
    return pl.pallas_call(
        _tc_mid_body,
        grid=(N // TN,),
        in_specs=[
            pl.BlockSpec((2, TN, H), lambda i: (0, i, 0)),
            pl.BlockSpec((TN, 1), lambda i: (i, 0)),
            pl.BlockSpec((TN, D), lambda i: (i, 0)),
            pl.BlockSpec((D, D), lambda i: (0, 0)),
            pl.BlockSpec((D, D), lambda i: (0, 0)),
            pl.BlockSpec((1, D), lambda i: (0, 0)),
        ],
        out_specs=[
            pl.BlockSpec((2, TN, H), lambda i: (0, i, 0)),
            pl.BlockSpec((TN, D), lambda i: (i, 0)),
        ],
        out_shape=[
            jax.ShapeDtypeStruct((2, N, H), jnp.float32),
            jax.ShapeDtypeStruct((N, D), jnp.float32),
        ],
    )(agg, cnt, r, wl, wr, b.reshape(1, D))


def _tc_post_body(agg_ref, cnt_ref, r_ref, o_ref):
    recip = 1.0 / jnp.maximum(cnt_ref[...], 1.0)
    o_ref[...] = (jnp.concatenate([agg_ref[0], agg_ref[1]], axis=1) * recip
                  + r_ref[...])


def _tc_post(agg, cnt, r):
    return pl.pallas_call(
        _tc_post_body,
        grid=(N // TN,),
        in_specs=[
            pl.BlockSpec((2, TN, H), lambda i: (0, i, 0)),
            pl.BlockSpec((TN, 1), lambda i: (i, 0)),
            pl.BlockSpec((TN, D), lambda i: (i, 0)),
        ],
        out_specs=pl.BlockSpec((TN, D), lambda i: (i, 0)),
        out_shape=jax.ShapeDtypeStruct((N, D), jnp.float32),
    )(agg, cnt, r)


def _make_sc_agg(with_cnt: bool):
    mesh = plsc.VectorSubcoreMesh(core_axis_name="c", subcore_axis_name="s")
    out_type = [jax.ShapeDtypeStruct((2, N, H), jnp.float32)]
    if with_cnt:
        out_type.append(jax.ShapeDtypeStruct((N,), jnp.float32))

    def body(y_hbm, srcs_hbm, dsts_hbm, agg_hbm, *rest):
        if with_cnt:
            cnt_hbm = rest[0]
            (src_v, dst_v, rows_v, ones_v, cnt_v, acc, cnt_acc,
             gsem, gsem2) = rest[1:]
        else:
            (src_v, dst_v, rows_v, ones_v, cnt_v, acc, cnt_acc,
             gsem, gsem2) = rest

        c = lax.axis_index("c")
        s = lax.axis_index("s")

        zv = jnp.zeros((16,), jnp.float32)

        def zbody(i, _):
            for k in range(H // 16):
                rows_v[0, i, pl.ds(k * 16, 16)] = zv
            return 0

        lax.fori_loop(0, CHUNK, zbody, 0)
        for t in range(ZROWS // CHUNK):
            pltpu.sync_copy(rows_v.at[0],
                            acc.at[pl.ds(s * ZROWS + t * CHUNK, CHUNK)])
        if with_cnt:
            for k in range(1024 // 16):
                cnt_v[pl.ds(k * 16, 16)] = zv

            @pl.when(c == 0)
            def _():
                pltpu.sync_copy(cnt_v.at[pl.ds(0, NPAD // NTILES)],
                                cnt_acc.at[pl.ds(s * (NPAD // NTILES),
                                                 NPAD // NTILES)])

        pltpu.sync_copy(srcs_hbm.at[s], src_v)
        pltpu.sync_copy(dsts_hbm.at[s], dst_v)

        off = c * N

        def off_body(i, _):
            sl = pl.ds(i * 16, 16)
            src_v[sl] = src_v[sl] + off
            return 0

        lax.fori_loop(0, EPT // 16, off_body, 0)

        if with_cnt:
            for k in range(CHUNK // 16):
                ones_v[pl.ds(k * 16, 16)] = jnp.full((16,), 1.0, jnp.float32)

        plsc.subcore_barrier()

        def g_desc(j, b, sem):
            idx = src_v.at[pl.ds(j * CHUNK, CHUNK)]
            return pltpu.make_async_copy(y_hbm.at[idx], rows_v.at[b], sem)

        g_desc(0, 0, gsem).start()

        def pair_body(jj, _):
            j0 = 2 * jj
            g_desc(j0, 0, gsem).wait()
            g_desc(j0 + 1, 1, gsem2).start()
            pltpu.sync_copy(rows_v.at[0], acc.at[dst_v.at[j0]], add=True)
            g_desc(j0 + 1, 1, gsem2).wait()

            @pl.when(jj + 1 < NCH // 2)
            def _():
                g_desc(j0 + 2, 0, gsem).start()

            pltpu.sync_copy(rows_v.at[1], acc.at[dst_v.at[j0 + 1]], add=True)
            return 0

        lax.fori_loop(0, NCH // 2, pair_body, 0)

        if with_cnt:
            @pl.when(c == 0)
            def _():
                def cnt_body(j, _):
                    pltpu.sync_copy(ones_v, cnt_acc.at[dst_v.at[j]], add=True)
                    return 0
                lax.fori_loop(0, NCH, cnt_body, 0)

        plsc.subcore_barrier()

        @pl.when(s < 10)
        def _():
            pltpu.sync_copy(acc.at[pl.ds(s * OROWS, OROWS)],
                            agg_hbm.at[c, pl.ds(s * OROWS, OROWS)])
        if with_cnt:
            @pl.when(jnp.logical_and(c == 0, s < 10))
            def _():
                pltpu.sync_copy(cnt_acc.at[pl.ds(s * 1000, 1000)],
                                cnt_v.at[pl.ds(0, 1000)])
                pltpu.sync_copy(cnt_v.at[pl.ds(0, 1000)],
                                cnt_hbm.at[pl.ds(s * 1000, 1000)])

    return pl.kernel(
        body,
        out_type=out_type,
        mesh=mesh,
        scratch_types=[
            pltpu.VMEM((EPT,), jnp.int32),
            pltpu.VMEM((NCH, CHUNK), jnp.int32),
            pltpu.VMEM((2, CHUNK, H), jnp.float32),
            pltpu.VMEM((CHUNK,), jnp.float32),
            pltpu.VMEM((1024,), jnp.float32),
            pltpu.VMEM_SHARED((NPAD, H), jnp.float32),
            pltpu.VMEM_SHARED((NPAD,), jnp.float32),
            pltpu.SemaphoreType.DMA,
            pltpu.SemaphoreType.DMA,
        ],
    )


_sc_agg_cnt = _make_sc_agg(True)
_sc_agg = _make_sc_agg(False)


def kernel(x, edge_index, W1_l, b1, W1_r, W2_l, b2, W2_r):
    src = edge_index[0].astype(jnp.int32)
    dst = edge_index[1].astype(jnp.int32)
    pad = EPAD - E
    src_p = jnp.concatenate([src, jnp.zeros((pad,), jnp.int32)])
    dst_p = jnp.concatenate([dst, jnp.full((pad,), N, jnp.int32)])
    srcs = src_p.reshape(NTILES, EPT)
    dsts = dst_p.reshape(NTILES, NCH, CHUNK)

    y1, r1 = _tc_pre(x, W1_l, W1_r, b1)
    agg1, cnt = _sc_agg_cnt(y1.reshape(2 * N, H), srcs, dsts)
    cnt2 = cnt.reshape(N, 1)
    y2, r2 = _tc_mid(agg1, cnt2, r1, W2_l, W2_r, b2)
    (agg2,) = _sc_agg(y2.reshape(2 * N, H), srcs, dsts)
    return _tc_post(agg2, cnt2, r2)
